# baseline (speedup 1.0000x reference)
import functools

import jax
import jax.numpy as jnp
from jax import lax
from jax.experimental import pallas as pl
from jax.experimental.pallas import tpu as pltpu
from jax.experimental.pallas import tpu_sc as plsc

BATCH = 16384
TOPIC_DIM = 128
SUBTOPIC_DIM = 64
OUT_DIM = TOPIC_DIM + SUBTOPIC_DIM
CHUNK = 128
TC_BLOCK = 512


@functools.cache
def _build_gather():
    info = plsc.get_sparse_core_info()
    num_workers = info.num_cores * info.num_subcores
    rows_per_worker = BATCH // num_workers
    n_chunks = rows_per_worker // CHUNK
    mesh = plsc.VectorSubcoreMesh(core_axis_name="c", subcore_axis_name="s")

    scratch = [
        pltpu.VMEM((n_chunks, CHUNK), jnp.int32),
        pltpu.VMEM((n_chunks, CHUNK), jnp.int32),
        pltpu.VMEM((rows_per_worker, TOPIC_DIM), jnp.float32),
        pltpu.VMEM((rows_per_worker, SUBTOPIC_DIM), jnp.float32),
        pltpu.SemaphoreType.DMA,
        pltpu.SemaphoreType.DMA,
        pltpu.SemaphoreType.DMA,
    ]

    @functools.partial(
        pl.kernel,
        mesh=mesh,
        out_type=(
            jax.ShapeDtypeStruct((BATCH, TOPIC_DIM), jnp.float32),
            jax.ShapeDtypeStruct((BATCH, 2 * SUBTOPIC_DIM), jnp.float32),
        ),
        scratch_types=scratch,
        compiler_params=pltpu.CompilerParams(use_tc_tiling_on_sc=False),
    )
    def enc(t_idx_hbm, s_idx_hbm, t_tab_hbm, s_tab_hbm, t_im_hbm, s_im_hbm,
            t_idx_v, s_idx_v, t_rows, s_rows, t_sem, s_sem, w_sem):
        wid = lax.axis_index("s") * info.num_cores + lax.axis_index("c")
        base = wid * rows_per_worker
        idx_row0 = wid * n_chunks

        ti_cp = pltpu.async_copy(
            t_idx_hbm.at[pl.ds(idx_row0, n_chunks)], t_idx_v, t_sem)
        si_cp = pltpu.async_copy(
            s_idx_hbm.at[pl.ds(idx_row0, n_chunks)], s_idx_v, s_sem)

        t_cps = []
        s_cps = []
        ti_cp.wait()
        si_cp.wait()
        for c in range(n_chunks):
            rows = pl.ds(c * CHUNK, CHUNK)
            t_cps.append(pltpu.async_copy(
                t_tab_hbm.at[t_idx_v.at[c]], t_rows.at[rows], t_sem))
            s_cps.append(pltpu.async_copy(
                s_tab_hbm.at[s_idx_v.at[c]], s_rows.at[rows], s_sem))

        half = rows_per_worker // 2
        w_cps = []
        for h in range(2):
            rows_h = pl.ds(h * half, half)
            out_rows = pl.ds(base + h * half, half)
            for cp in t_cps[2 * h:2 * h + 2]:
                cp.wait()
            w_cps.append(pltpu.async_copy(
                t_rows.at[rows_h], t_im_hbm.at[out_rows], w_sem))
            for cp in s_cps[2 * h:2 * h + 2]:
                cp.wait()
            w_cps.append(pltpu.async_copy(
                s_rows.at[rows_h],
                s_im_hbm.at[out_rows, pl.ds(0, SUBTOPIC_DIM)], w_sem))
        for cp in w_cps:
            cp.wait()

    return enc


def _tc_assemble(t_ref, s_ref, o_ref):
    o_ref[:, 0:TOPIC_DIM] = t_ref[...]
    o_ref[:, TOPIC_DIM:OUT_DIM] = s_ref[:, 0:SUBTOPIC_DIM]


@functools.cache
def _build_assemble():
    return pl.pallas_call(
        _tc_assemble,
        grid=(BATCH // TC_BLOCK,),
        in_specs=[
            pl.BlockSpec((TC_BLOCK, TOPIC_DIM), lambda i: (i, 0)),
            pl.BlockSpec((TC_BLOCK, 2 * SUBTOPIC_DIM), lambda i: (i, 0)),
        ],
        out_specs=pl.BlockSpec((TC_BLOCK, OUT_DIM), lambda i: (i, 0)),
        out_shape=jax.ShapeDtypeStruct((BATCH, OUT_DIM), jnp.float32),
    )


def kernel(topic, subtopic, topic_table, subtopic_table):
    enc = _build_gather()
    n_rows = BATCH // CHUNK
    t_im, s_im = enc(topic.astype(jnp.int32).reshape(n_rows, CHUNK),
                     subtopic.astype(jnp.int32).reshape(n_rows, CHUNK),
                     topic_table, subtopic_table)
    return _build_assemble()(t_im, s_im)


# speedup vs baseline: 1.0001x; 1.0001x over previous
import functools

import jax
import jax.numpy as jnp
from jax import lax
from jax.experimental import pallas as pl
from jax.experimental.pallas import tpu as pltpu
from jax.experimental.pallas import tpu_sc as plsc

BATCH = 16384
TOPIC_DIM = 128
SUBTOPIC_DIM = 64
OUT_DIM = TOPIC_DIM + SUBTOPIC_DIM
CHUNK = 128
TC_BLOCK = 512


@functools.cache
def _build_gather():
    info = plsc.get_sparse_core_info()
    num_workers = info.num_cores * info.num_subcores
    rows_per_worker = BATCH // num_workers
    n_chunks = rows_per_worker // CHUNK
    mesh = plsc.VectorSubcoreMesh(core_axis_name="c", subcore_axis_name="s")

    scratch = [
        pltpu.VMEM((rows_per_worker,), jnp.int32),
        pltpu.VMEM((rows_per_worker,), jnp.int32),
        pltpu.VMEM((rows_per_worker, TOPIC_DIM), jnp.float32),
        pltpu.VMEM((rows_per_worker, SUBTOPIC_DIM), jnp.float32),
        pltpu.SemaphoreType.DMA,
        pltpu.SemaphoreType.DMA,
        pltpu.SemaphoreType.DMA,
    ]

    @functools.partial(
        pl.kernel,
        mesh=mesh,
        out_type=(
            jax.ShapeDtypeStruct((BATCH, TOPIC_DIM), jnp.float32),
            jax.ShapeDtypeStruct((BATCH, 2 * SUBTOPIC_DIM), jnp.float32),
        ),
        scratch_types=scratch,
        compiler_params=pltpu.CompilerParams(use_tc_tiling_on_sc=False),
    )
    def enc(t_idx_hbm, s_idx_hbm, t_tab_hbm, s_tab_hbm, t_im_hbm, s_im_hbm,
            t_idx_v, s_idx_v, t_rows, s_rows, t_sem, s_sem, w_sem):
        wid = lax.axis_index("s") * info.num_cores + lax.axis_index("c")
        base = wid * rows_per_worker

        ti_cp = pltpu.async_copy(
            t_idx_hbm.at[pl.ds(base, rows_per_worker)], t_idx_v, t_sem)
        si_cp = pltpu.async_copy(
            s_idx_hbm.at[pl.ds(base, rows_per_worker)], s_idx_v, s_sem)

        t_cps = []
        s_cps = []
        ti_cp.wait()
        si_cp.wait()
        for c in range(n_chunks):
            rows = pl.ds(c * CHUNK, CHUNK)
            t_cps.append(pltpu.async_copy(
                t_tab_hbm.at[t_idx_v.at[rows]], t_rows.at[rows], t_sem))
            s_cps.append(pltpu.async_copy(
                s_tab_hbm.at[s_idx_v.at[rows]], s_rows.at[rows], s_sem))

        half = rows_per_worker // 2
        w_cps = []
        for h in range(2):
            rows_h = pl.ds(h * half, half)
            out_rows = pl.ds(base + h * half, half)
            for cp in t_cps[2 * h:2 * h + 2]:
                cp.wait()
            w_cps.append(pltpu.async_copy(
                t_rows.at[rows_h], t_im_hbm.at[out_rows], w_sem))
            for cp in s_cps[2 * h:2 * h + 2]:
                cp.wait()
            w_cps.append(pltpu.async_copy(
                s_rows.at[rows_h],
                s_im_hbm.at[out_rows, pl.ds(0, SUBTOPIC_DIM)], w_sem))
        for cp in w_cps:
            cp.wait()

    return enc


def _tc_assemble(t_ref, s_ref, o_ref):
    o_ref[:, 0:TOPIC_DIM] = t_ref[...]
    o_ref[:, TOPIC_DIM:OUT_DIM] = s_ref[:, 0:SUBTOPIC_DIM]


@functools.cache
def _build_assemble():
    return pl.pallas_call(
        _tc_assemble,
        grid=(BATCH // TC_BLOCK,),
        in_specs=[
            pl.BlockSpec((TC_BLOCK, TOPIC_DIM), lambda i: (i, 0)),
            pl.BlockSpec((TC_BLOCK, 2 * SUBTOPIC_DIM), lambda i: (i, 0)),
        ],
        out_specs=pl.BlockSpec((TC_BLOCK, OUT_DIM), lambda i: (i, 0)),
        out_shape=jax.ShapeDtypeStruct((BATCH, OUT_DIM), jnp.float32),
    )


def kernel(topic, subtopic, topic_table, subtopic_table):
    enc = _build_gather()
    t_im, s_im = enc(topic.astype(jnp.int32), subtopic.astype(jnp.int32),
                     topic_table, subtopic_table)
    return _build_assemble()(t_im, s_im)


# speedup vs baseline: 1.0038x; 1.0037x over previous
import functools

import jax
import jax.numpy as jnp
from jax import lax
from jax.experimental import pallas as pl
from jax.experimental.pallas import tpu as pltpu
from jax.experimental.pallas import tpu_sc as plsc

BATCH = 16384
TOPIC_DIM = 128
SUBTOPIC_DIM = 64
OUT_DIM = TOPIC_DIM + SUBTOPIC_DIM
CHUNK = 128


@functools.cache
def _build():
    info = plsc.get_sparse_core_info()
    num_workers = info.num_cores * info.num_subcores
    rows_per_worker = BATCH // num_workers
    n_chunks = rows_per_worker // CHUNK
    mesh = plsc.VectorSubcoreMesh(core_axis_name="c", subcore_axis_name="s")

    scratch = [
        pltpu.VMEM((rows_per_worker,), jnp.int32),
        pltpu.VMEM((rows_per_worker,), jnp.int32),
        pltpu.VMEM((rows_per_worker, TOPIC_DIM), jnp.float32),
        [pltpu.VMEM((CHUNK, 2 * SUBTOPIC_DIM), jnp.float32)] * 2,
        pltpu.SemaphoreType.DMA,
        [pltpu.SemaphoreType.DMA] * 2,
        pltpu.SemaphoreType.DMA,
        [pltpu.SemaphoreType.DMA] * 2,
    ]

    @functools.partial(
        pl.kernel,
        mesh=mesh,
        out_type=jax.ShapeDtypeStruct((BATCH, OUT_DIM), jnp.float32),
        scratch_types=scratch,
        compiler_params=pltpu.CompilerParams(use_tc_tiling_on_sc=False),
    )
    def enc(t_idx_hbm, s_idx_hbm, t_tab_hbm, s_dup_hbm, out_hbm,
            t_idx_v, s_idx_v, t_rows, s_bufs, t_sem, s_sems, tw_sem, sw_sems):
        wid = lax.axis_index("s") * info.num_cores + lax.axis_index("c")
        base = wid * rows_per_worker

        ti_cp = pltpu.async_copy(
            t_idx_hbm.at[pl.ds(base, rows_per_worker)], t_idx_v, t_sem)
        si_cp = pltpu.async_copy(
            s_idx_hbm.at[pl.ds(base, rows_per_worker)], s_idx_v, s_sems[0])
        ti_cp.wait()
        si_cp.wait()

        t_cps = []
        for c in range(n_chunks):
            rows = pl.ds(c * CHUNK, CHUNK)
            t_cps.append(pltpu.async_copy(
                t_tab_hbm.at[t_idx_v.at[rows]], t_rows.at[rows], t_sem))

        s_cps = [None] * n_chunks
        sw_cps = [None] * n_chunks
        for c in range(2):
            s_cps[c] = pltpu.async_copy(
                s_dup_hbm.at[s_idx_v.at[pl.ds(c * CHUNK, CHUNK)]],
                s_bufs[c], s_sems[c])

        for c in range(n_chunks):
            b = c % 2
            s_cps[c].wait()
            sw_cps[c] = pltpu.async_copy(
                s_bufs[b].at[:, pl.ds(0, SUBTOPIC_DIM)],
                out_hbm.at[pl.ds(base + c * CHUNK, CHUNK),
                           pl.ds(TOPIC_DIM, SUBTOPIC_DIM)],
                sw_sems[b])
            if c + 2 < n_chunks:
                sw_cps[c].wait()
                s_cps[c + 2] = pltpu.async_copy(
                    s_dup_hbm.at[s_idx_v.at[pl.ds((c + 2) * CHUNK, CHUNK)]],
                    s_bufs[b], s_sems[b])

        for cp in t_cps:
            cp.wait()
        tw_cp = pltpu.async_copy(
            t_rows,
            out_hbm.at[pl.ds(base, rows_per_worker), pl.ds(0, TOPIC_DIM)],
            tw_sem)

        tw_cp.wait()
        sw_cps[n_chunks - 2].wait()
        sw_cps[n_chunks - 1].wait()

    return enc


def kernel(topic, subtopic, topic_table, subtopic_table):
    enc = _build()
    dup_table = jnp.concatenate([subtopic_table, subtopic_table], axis=1)
    return enc(topic.astype(jnp.int32), subtopic.astype(jnp.int32),
               topic_table, dup_table)


# speedup vs baseline: 1.0723x; 1.0682x over previous
import functools

import jax
import jax.numpy as jnp
from jax import lax
from jax.experimental import pallas as pl
from jax.experimental.pallas import tpu as pltpu
from jax.experimental.pallas import tpu_sc as plsc

BATCH = 16384
TOPIC_DIM = 128
SUBTOPIC_DIM = 64
OUT_DIM = TOPIC_DIM + SUBTOPIC_DIM
CHUNK = 128


@functools.cache
def _build():
    info = plsc.get_sparse_core_info()
    num_workers = info.num_cores * info.num_subcores
    rows_per_worker = BATCH // num_workers
    n_chunks = rows_per_worker // CHUNK
    mesh = plsc.VectorSubcoreMesh(core_axis_name="c", subcore_axis_name="s")

    scratch = [
        pltpu.VMEM((rows_per_worker,), jnp.int32),
        pltpu.VMEM((rows_per_worker,), jnp.int32),
        pltpu.VMEM((rows_per_worker, TOPIC_DIM), jnp.float32),
        pltpu.VMEM((rows_per_worker, SUBTOPIC_DIM), jnp.float32),
        pltpu.SemaphoreType.DMA,
        pltpu.SemaphoreType.DMA,
        pltpu.SemaphoreType.DMA,
    ]

    @functools.partial(
        pl.kernel,
        mesh=mesh,
        out_type=jax.ShapeDtypeStruct((BATCH, OUT_DIM), jnp.float32),
        scratch_types=scratch,
        compiler_params=pltpu.CompilerParams(use_tc_tiling_on_sc=False),
    )
    def enc(t_idx_hbm, s_idx_hbm, t_tab_hbm, s_tab_hbm, out_hbm,
            t_idx_v, s_idx_v, t_rows, s_rows, t_sem, s_sem, w_sem):
        wid = lax.axis_index("s") * info.num_cores + lax.axis_index("c")
        base = wid * rows_per_worker

        pltpu.sync_copy(t_idx_hbm.at[pl.ds(base, rows_per_worker)], t_idx_v)
        pltpu.sync_copy(s_idx_hbm.at[pl.ds(base, rows_per_worker)], s_idx_v)

        t_cps = []
        s_cps = []
        for c in range(n_chunks):
            rows = pl.ds(c * CHUNK, CHUNK)
            t_cps.append(pltpu.async_copy(
                t_tab_hbm.at[t_idx_v.at[rows]], t_rows.at[rows], t_sem))
            s_cps.append(pltpu.async_copy(
                s_tab_hbm.at[s_idx_v.at[rows]], s_rows.at[rows], s_sem))

        for cp in t_cps:
            cp.wait()
        w1 = pltpu.async_copy(
            t_rows,
            out_hbm.at[pl.ds(base, rows_per_worker), pl.ds(0, TOPIC_DIM)],
            w_sem)
        for cp in s_cps:
            cp.wait()
        w2 = pltpu.async_copy(
            s_rows,
            out_hbm.at[pl.ds(base, rows_per_worker),
                       pl.ds(TOPIC_DIM, SUBTOPIC_DIM)],
            w_sem)
        w1.wait()
        w2.wait()

    return enc


def kernel(topic, subtopic, topic_table, subtopic_table):
    enc = _build()
    return enc(topic.astype(jnp.int32), subtopic.astype(jnp.int32),
               topic_table, subtopic_table)
